# baseline (speedup 1.0000x reference)
import jax
import jax.numpy as jnp
from jax import lax
from jax.experimental import pallas as pl
from jax.experimental.pallas import tpu as pltpu
from jax.experimental.pallas import tpu_sc as plsc

_N = 10000
_E = 320000
_D = 128
_CH = 128
_NC = 2
_NS = 16
_NW = _NC * _NS
_NB = _E // _CH
_BPW = _NB // _NW
_EXTRA = _NB % _NW
_BMAX = _BPW + 1
_ZCH = 80
_NZ = _N // _ZCH
_NBUF = 3


def _sc_scatter(edge_index, edge_feat):
    mesh = plsc.VectorSubcoreMesh(core_axis_name="c", subcore_axis_name="s")

    def body(eidx_hbm, edge_hbm, out_hbm,
             ibuf0, ibuf1, ibuf2, ebuf0, ebuf1, ebuf2, agg_shared,
             gsem0, gsem1, gsem2, isem0, isem1, isem2,
             ssem0, ssem1, ssem2):
        cid = lax.axis_index("c")
        sid = lax.axis_index("s")
        wid = cid * _NS + sid
        eph = _EXTRA // _NC
        nblk = _BPW + jnp.where(sid < eph, 1, 0)
        blk0 = _BPW * wid + cid * eph + jnp.minimum(sid, eph)

        ibufs = (ibuf0, ibuf1, ibuf2)
        ebufs = (ebuf0, ebuf1, ebuf2)
        gsems = (gsem0, gsem1, gsem2)
        isems = (isem0, isem1, isem2)
        ssems = (ssem0, ssem1, ssem2)

        def start_gather(i, b):
            off = (blk0 + i) * _CH
            pltpu.async_copy(edge_hbm.at[pl.ds(off, _CH)], ebufs[b],
                             gsems[b])
            pltpu.async_copy(eidx_hbm.at[0, pl.ds(off, _CH)], ibufs[b],
                             isems[b])

        def wait_gather(b):
            pltpu.make_async_copy(
                edge_hbm.at[pl.ds(0, _CH)], ebufs[b], gsems[b]).wait()
            pltpu.make_async_copy(
                eidx_hbm.at[0, pl.ds(0, _CH)], ibufs[b], isems[b]).wait()

        def start_scatter(b):
            pltpu.async_copy(ebufs[b], agg_shared.at[ibufs[b]], ssems[b],
                             add=True)

        def wait_scatter(b):
            pltpu.make_async_copy(
                ebufs[b], agg_shared.at[ibufs[b]], ssems[b]).wait()

        for pb in (1, 2):
            start_gather(pb, pb)

        zv = jnp.zeros((16,), jnp.float32)

        def zrow(i, carry):
            def zcol(j, c2):
                ebuf0[i, pl.ds(j * 16, 16)] = zv
                return c2
            return lax.fori_loop(0, _D // 16, zcol, carry)

        lax.fori_loop(0, _ZCH, zrow, 0)

        def zchunk(k, carry):
            c = sid + k * _NS

            @pl.when(c < _NZ)
            def _():
                pltpu.sync_copy(ebuf0.at[pl.ds(0, _ZCH)],
                                agg_shared.at[pl.ds(c * _ZCH, _ZCH)])

            return carry

        lax.fori_loop(0, (_NZ + _NS - 1) // _NS, zchunk, 0)
        start_gather(0, 0)
        plsc.subcore_barrier()

        def group(g, carry):
            base = _NBUF * g
            for b in range(_NBUF):
                i = base + b

                @pl.when(i < nblk)
                def _(i=i, b=b):
                    wait_gather(b)
                    start_scatter(b)

            for b in range(_NBUF):
                i = base + b

                @pl.when(i < nblk)
                def _(i=i, b=b):
                    wait_scatter(b)

                    @pl.when(i + _NBUF < nblk)
                    def _():
                        start_gather(i + _NBUF, b)

            return carry

        lax.fori_loop(0, (_BMAX + _NBUF - 1) // _NBUF, group, 0)
        plsc.subcore_barrier()

        def dump(k, carry):
            c = sid + k * _NS

            @pl.when(c < _NZ)
            def _():
                pltpu.sync_copy(agg_shared.at[pl.ds(c * _ZCH, _ZCH)],
                                out_hbm.at[cid, pl.ds(c * _ZCH, _ZCH)])

            return carry

        lax.fori_loop(0, (_NZ + _NS - 1) // _NS, dump, 0)

    return pl.kernel(
        body,
        out_type=jax.ShapeDtypeStruct((_NC, _N, _D), jnp.float32),
        mesh=mesh,
        scratch_types=[
            pltpu.VMEM((_CH,), jnp.int32),
            pltpu.VMEM((_CH,), jnp.int32),
            pltpu.VMEM((_CH,), jnp.int32),
            pltpu.VMEM((_CH, _D), jnp.float32),
            pltpu.VMEM((_CH, _D), jnp.float32),
            pltpu.VMEM((_CH, _D), jnp.float32),
            pltpu.VMEM_SHARED((_N, _D), jnp.float32),
        ] + [pltpu.SemaphoreType.DMA] * 9,
    )(edge_index, edge_feat)


_BR = 1000


def _node_half(node_feat, W, b):
    def body(nf, wt, bb, o):
        o[...] = jnp.dot(nf[...], wt[0],
                         preferred_element_type=jnp.float32) + bb[...]

    w3 = W.reshape(2, _D, _D)
    b2 = b.reshape(1, _D)
    return pl.pallas_call(
        body,
        grid=(_N // _BR,),
        in_specs=[
            pl.BlockSpec((_BR, _D), lambda i: (i, 0)),
            pl.BlockSpec((1, _D, _D), lambda i: (0, 0, 0)),
            pl.BlockSpec((1, _D), lambda i: (0, 0)),
        ],
        out_specs=pl.BlockSpec((_BR, _D), lambda i: (i, 0)),
        out_shape=jax.ShapeDtypeStruct((_N, _D), jnp.float32),
    )(node_feat, w3, b2)


def _agg_half(tmp, partials, W):
    def body(tp, a0, a1, wb, o):
        agg = a0[0] + a1[0]
        o[...] = tp[...] + jnp.dot(agg, wb[0],
                                   preferred_element_type=jnp.float32)

    w3 = W.reshape(2, _D, _D)
    return pl.pallas_call(
        body,
        grid=(_N // _BR,),
        in_specs=[
            pl.BlockSpec((_BR, _D), lambda i: (i, 0)),
            pl.BlockSpec((1, _BR, _D), lambda i: (0, i, 0)),
            pl.BlockSpec((1, _BR, _D), lambda i: (1, i, 0)),
            pl.BlockSpec((1, _D, _D), lambda i: (1, 0, 0)),
        ],
        out_specs=pl.BlockSpec((_BR, _D), lambda i: (i, 0)),
        out_shape=jax.ShapeDtypeStruct((_N, _D), jnp.float32),
    )(tmp, partials, partials, w3)


def kernel(node_feat, edge_feat, edge_index, W, b):
    partials = _sc_scatter(edge_index, edge_feat)
    tmp = _node_half(node_feat, W, b)
    return _agg_half(tmp, partials, W)


# speedup vs baseline: 1.0242x; 1.0242x over previous
import jax
import jax.numpy as jnp
from jax import lax
from jax.experimental import pallas as pl
from jax.experimental.pallas import tpu as pltpu
from jax.experimental.pallas import tpu_sc as plsc

_N = 10000
_E = 320000
_D = 128
_CH = 128
_NC = 2
_NS = 16
_NW = _NC * _NS
_NB = _E // _CH
_BPW = _NB // _NW
_EXTRA = _NB % _NW
_BMAX = _BPW + 1
_ZCH = 80
_NZ = _N // _ZCH
_NBUF = 3


def _sc_scatter(edge_index, edge_feat):
    mesh = plsc.VectorSubcoreMesh(core_axis_name="c", subcore_axis_name="s")

    def body(eidx_hbm, edge_hbm, out_hbm,
             ibuf0, ibuf1, ibuf2, ebuf0, ebuf1, ebuf2, agg_shared,
             gsem0, gsem1, gsem2, isem0, isem1, isem2,
             ssem0, ssem1, ssem2):
        cid = lax.axis_index("c")
        sid = lax.axis_index("s")
        wid = cid * _NS + sid
        eph = _EXTRA // _NC
        nblk = _BPW + jnp.where(sid < eph, 1, 0)
        blk0 = _BPW * wid + cid * eph + jnp.minimum(sid, eph)

        ibufs = (ibuf0, ibuf1, ibuf2)
        ebufs = (ebuf0, ebuf1, ebuf2)
        gsems = (gsem0, gsem1, gsem2)
        isems = (isem0, isem1, isem2)
        ssems = (ssem0, ssem1, ssem2)

        def start_gather(i, b):
            off = (blk0 + i) * _CH
            pltpu.async_copy(edge_hbm.at[pl.ds(off, _CH)], ebufs[b],
                             gsems[b])
            pltpu.async_copy(eidx_hbm.at[0, pl.ds(off, _CH)], ibufs[b],
                             isems[b])

        def wait_gather(b):
            pltpu.make_async_copy(
                edge_hbm.at[pl.ds(0, _CH)], ebufs[b], gsems[b]).wait()
            pltpu.make_async_copy(
                eidx_hbm.at[0, pl.ds(0, _CH)], ibufs[b], isems[b]).wait()

        def start_scatter(b):
            pltpu.async_copy(ebufs[b], agg_shared.at[ibufs[b]], ssems[b],
                             add=True)

        def wait_scatter(b):
            pltpu.make_async_copy(
                ebufs[b], agg_shared.at[ibufs[b]], ssems[b]).wait()

        for pb in (1, 2):
            start_gather(pb, pb)

        zv = jnp.zeros((16,), jnp.float32)

        def zrow(i, carry):
            def zcol(j, c2):
                ebuf0[i, pl.ds(j * 16, 16)] = zv
                return c2
            return lax.fori_loop(0, _D // 16, zcol, carry)

        lax.fori_loop(0, _ZCH, zrow, 0)

        def zchunk(k, carry):
            c = sid + k * _NS

            @pl.when(c < _NZ)
            def _():
                pltpu.sync_copy(ebuf0.at[pl.ds(0, _ZCH)],
                                agg_shared.at[pl.ds(c * _ZCH, _ZCH)])

            return carry

        lax.fori_loop(0, (_NZ + _NS - 1) // _NS, zchunk, 0)
        start_gather(0, 0)
        plsc.subcore_barrier()

        def group(g, carry):
            base = _NBUF * g
            for b in range(_NBUF):
                i = base + b

                @pl.when(i < nblk)
                def _(i=i, b=b):
                    wait_gather(b)
                    start_scatter(b)

            for b in range(_NBUF):
                i = base + b

                @pl.when(i < nblk)
                def _(i=i, b=b):
                    wait_scatter(b)

                    @pl.when(i + _NBUF < nblk)
                    def _():
                        start_gather(i + _NBUF, b)

            return carry

        lax.fori_loop(0, (_BMAX + _NBUF - 1) // _NBUF, group, 0)
        plsc.subcore_barrier()

        def dump(k, carry):
            c = sid + k * _NS

            @pl.when(c < _NZ)
            def _():
                pltpu.sync_copy(agg_shared.at[pl.ds(c * _ZCH, _ZCH)],
                                out_hbm.at[cid, pl.ds(c * _ZCH, _ZCH)])

            return carry

        lax.fori_loop(0, (_NZ + _NS - 1) // _NS, dump, 0)

    return pl.kernel(
        body,
        out_type=jax.ShapeDtypeStruct((_NC, _N, _D), jnp.float32),
        mesh=mesh,
        scratch_types=[
            pltpu.VMEM((_CH,), jnp.int32),
            pltpu.VMEM((_CH,), jnp.int32),
            pltpu.VMEM((_CH,), jnp.int32),
            pltpu.VMEM((_CH, _D), jnp.float32),
            pltpu.VMEM((_CH, _D), jnp.float32),
            pltpu.VMEM((_CH, _D), jnp.float32),
            pltpu.VMEM_SHARED((_N, _D), jnp.float32),
        ] + [pltpu.SemaphoreType.DMA] * 9,
    )(edge_index, edge_feat)


_BR = 2000


def _node_half(node_feat, W, b):
    def body(nf, wt, bb, o):
        o[...] = jnp.dot(nf[...], wt[0],
                         preferred_element_type=jnp.float32) + bb[...]

    w3 = W.reshape(2, _D, _D)
    b2 = b.reshape(1, _D)
    return pl.pallas_call(
        body,
        grid=(_N // _BR,),
        in_specs=[
            pl.BlockSpec((_BR, _D), lambda i: (i, 0)),
            pl.BlockSpec((1, _D, _D), lambda i: (0, 0, 0)),
            pl.BlockSpec((1, _D), lambda i: (0, 0)),
        ],
        out_specs=pl.BlockSpec((_BR, _D), lambda i: (i, 0)),
        out_shape=jax.ShapeDtypeStruct((_N, _D), jnp.float32),
    )(node_feat, w3, b2)


def _agg_half(tmp, partials, W):
    def body(tp, a0, a1, wb, o):
        agg = a0[0] + a1[0]
        o[...] = tp[...] + jnp.dot(agg, wb[0],
                                   preferred_element_type=jnp.float32)

    w3 = W.reshape(2, _D, _D)
    return pl.pallas_call(
        body,
        grid=(_N // _BR,),
        in_specs=[
            pl.BlockSpec((_BR, _D), lambda i: (i, 0)),
            pl.BlockSpec((1, _BR, _D), lambda i: (0, i, 0)),
            pl.BlockSpec((1, _BR, _D), lambda i: (1, i, 0)),
            pl.BlockSpec((1, _D, _D), lambda i: (1, 0, 0)),
        ],
        out_specs=pl.BlockSpec((_BR, _D), lambda i: (i, 0)),
        out_shape=jax.ShapeDtypeStruct((_N, _D), jnp.float32),
    )(tmp, partials, partials, w3)


def kernel(node_feat, edge_feat, edge_index, W, b):
    partials = _sc_scatter(edge_index, edge_feat)
    tmp = _node_half(node_feat, W, b)
    return _agg_half(tmp, partials, W)
